# baseline (speedup 1.0000x reference)
import functools

import jax
import jax.numpy as jnp
from jax import lax
from jax.experimental import pallas as pl
from jax.experimental.pallas import tpu as pltpu
from jax.experimental.pallas import tpu_sc as plsc

VOCAB = 1000000
EMBED_DIM = 64
BATCH = 4096
SEQ = 200

_NC = 2
_NS = 16
_NW = _NC * _NS

_LANES = 128
_PAIRS = VOCAB // 2
_NTILE_A = 7808
_TPW = _NTILE_A // _NW
_TAIL_ROWS = VOCAB - _NTILE_A * 128
_TAIL_PAIRS = _TAIL_ROWS // 2

_NBUF = 4
_NSEQ = SEQ


def _relayout_kernel(table_t_hbm, tail_hbm, scratch_hbm,
                     src0, src1, dst0, dst1, tailv,
                     rsem0, rsem1, wsem0, wsem1):
    srcs = (src0, src1)
    dsts = (dst0, dst1)
    rsems = (rsem0, rsem1)
    wsems = (wsem0, wsem1)
    wid = lax.axis_index("s") * _NC + lax.axis_index("c")
    start = wid * _TPW
    iota = lax.iota(jnp.int32, 16)

    def rd(j, k):
        return pltpu.make_async_copy(
            table_t_hbm.at[:, pl.ds((start + j) * 128, 128)], srcs[k], rsems[k])

    def wr(j, k):
        return pltpu.make_async_copy(
            dsts[k], scratch_hbm.at[pl.ds((start + j) * 64, 64)], wsems[k])

    rd(0, 0).start()
    rd(1, 1).start()

    def body(t, carry):
        for k in range(2):
            j = 2 * t + k
            rd(j, k).wait()

            @pl.when(t > 0)
            def _():
                wr(j - 2, k).wait()

            def cc_body(cc, c2):
                for g in range(8):
                    d0 = (g % 4) * 16
                    c = 2 * cc + (g // 4)
                    val = plsc.load_gather(srcs[k], [d0 + iota,
                                                     jnp.full((16,), 0, jnp.int32) + c])
                    dsts[k][cc, pl.ds(g * 16, 16)] = val
                return c2

            lax.fori_loop(0, 64, cc_body, 0)
            wr(j, k).start()

            @pl.when(t < (_TPW // 2) - 1)
            def _():
                rd(j + 2, k).start()
        return carry

    lax.fori_loop(0, _TPW // 2, body, 0)
    wr(_TPW - 2, 0).wait()
    wr(_TPW - 1, 1).wait()

    @pl.when(wid == _NW - 1)
    def _():
        pltpu.sync_copy(tail_hbm, tailv)
        pltpu.sync_copy(tailv, scratch_hbm.at[pl.ds(_NTILE_A * 64, _TAIL_PAIRS)])


def _gather_kernel(scratch_hbm, idxt_hbm, out_hbm,
                   idx_v, i0, i1, i2, i3, g0, g1, g2, g3, t0, t1,
                   gs0, gs1, gs2, gs3, ts0, ts1):
    idx2 = (i0, i1, i2, i3)
    gbuf = (g0, g1, g2, g3)
    tbuf = (t0, t1)
    gsems = (gs0, gs1, gs2, gs3)
    tsems = (ts0, ts1)
    wid = lax.axis_index("s") * _NC + lax.axis_index("c")
    iota = lax.iota(jnp.int32, 16)

    pltpu.sync_copy(idxt_hbm.at[:, pl.ds(wid * 128, 128)], idx_v)

    def prep_and_fire(i, b):
        for c in range(8):
            idx2[b][pl.ds(16 * c, 16)] = lax.shift_right_logical(
                idx_v[i, pl.ds(16 * c, 16)], 1)
        pltpu.make_async_copy(scratch_hbm.at[idx2[b]], gbuf[b], gsems[b]).start()

    def out_tile(i, tb):
        for p in range(8):
            pltpu.make_async_copy(tbuf[tb].at[pl.ds(8 * p, 8), :],
                                  out_hbm.at[i, p, wid], tsems[tb]).start()

    def drain_out(i, tb):
        for p in range(8):
            pltpu.make_async_copy(tbuf[tb].at[pl.ds(8 * p, 8), :],
                                  out_hbm.at[i, p, wid], tsems[tb]).wait()

    for b in range(_NBUF):
        prep_and_fire(b, b)

    def body(t, carry):
        for b in range(_NBUF):
            i = _NBUF * t + b
            tb = b % 2
            pltpu.make_async_copy(scratch_hbm.at[idx2[b]], gbuf[b], gsems[b]).wait()

            if b >= 2:
                drain_out(i - 2, tb)
            else:
                @pl.when(t > 0)
                def _():
                    drain_out(i - 2, tb)

            halves = []
            for g in range(8):
                halves.append(
                    lax.shift_left(jnp.bitwise_and(idx_v[i, pl.ds(16 * g, 16)], 1), 6))

            def d_body(d, c2):
                for g in range(8):
                    val = plsc.load_gather(gbuf[b], [16 * g + iota, halves[g] + d])
                    tbuf[tb][d, pl.ds(16 * g, 16)] = val
                return c2

            lax.fori_loop(0, EMBED_DIM, d_body, 0)
            out_tile(i, tb)

            @pl.when(t < (_NSEQ // _NBUF) - 1)
            def _():
                prep_and_fire(i + _NBUF, b)
        return carry

    lax.fori_loop(0, _NSEQ // _NBUF, body, 0)
    drain_out(_NSEQ - 2, 0)
    drain_out(_NSEQ - 1, 1)


@jax.jit
def kernel(input_sentence, table):
    mesh = plsc.VectorSubcoreMesh(core_axis_name="c", subcore_axis_name="s")
    cparams = pltpu.CompilerParams(use_tc_tiling_on_sc=True, needs_layout_passes=False)

    table_t = table.T
    tail = table[_NTILE_A * 128:].reshape(_TAIL_PAIRS, _LANES)
    idxt = input_sentence.T.astype(jnp.int32)

    scratch = pl.kernel(
        _relayout_kernel,
        out_type=jax.ShapeDtypeStruct((_PAIRS, _LANES), jnp.float32),
        mesh=mesh,
        scratch_types=(
            [pltpu.VMEM((EMBED_DIM, _LANES), jnp.float32) for _ in range(4)]
            + [pltpu.VMEM((_TAIL_PAIRS, _LANES), jnp.float32)]
            + [pltpu.SemaphoreType.DMA for _ in range(4)]
        ),
        compiler_params=cparams,
    )(table_t, tail)

    out5d = pl.kernel(
        _gather_kernel,
        out_type=jax.ShapeDtypeStruct((SEQ, 8, 32, 8, _LANES), jnp.float32),
        mesh=mesh,
        scratch_types=(
            [pltpu.VMEM((SEQ, _LANES), jnp.int32)]
            + [pltpu.VMEM((_LANES,), jnp.int32) for _ in range(_NBUF)]
            + [pltpu.VMEM((_LANES, _LANES), jnp.float32) for _ in range(_NBUF)]
            + [pltpu.VMEM((EMBED_DIM, _LANES), jnp.float32) for _ in range(2)]
            + [pltpu.SemaphoreType.DMA for _ in range(_NBUF + 2)]
        ),
        compiler_params=cparams,
    )(scratch, idxt)

    return out5d.transpose(2, 4, 0, 1, 3).reshape(BATCH, SEQ, EMBED_DIM)


# speedup vs baseline: 1.7848x; 1.7848x over previous
import functools

import jax
import jax.numpy as jnp
from jax import lax
from jax.experimental import pallas as pl
from jax.experimental.pallas import tpu as pltpu
from jax.experimental.pallas import tpu_sc as plsc

VOCAB = 1000000
EMBED_DIM = 64
BATCH = 4096
SEQ = 200

_NC = 2
_NS = 16
_NW = _NC * _NS

_LANES = 128
_PAIRS = VOCAB // 2
_NTILE_A = 7808
_TPW = _NTILE_A // _NW
_TAIL_ROWS = VOCAB - _NTILE_A * 128
_TAIL_PAIRS = _TAIL_ROWS // 2

_NBUF = 4
_NSEQ = SEQ


def _relayout_kernel(table_t_hbm, tail_hbm, scratch_hbm,
                     src0, src1, dst0, dst1, tailv,
                     rsem0, rsem1, wsem0, wsem1):
    srcs = (src0, src1)
    dsts = (dst0, dst1)
    rsems = (rsem0, rsem1)
    wsems = (wsem0, wsem1)
    wid = lax.axis_index("s") * _NC + lax.axis_index("c")
    start = wid * _TPW
    iota = lax.iota(jnp.int32, 16)

    cA = tuple((16 * g + iota) >> 1 for g in range(8))
    cB = tuple(((16 * g + iota) & 1) * 64 for g in range(8))

    def rd(j, k):
        return pltpu.make_async_copy(
            table_t_hbm.at[:, pl.ds((start + j) * 128, 128)], srcs[k], rsems[k])

    def wr(j, k):
        return pltpu.make_async_copy(
            dsts[k], scratch_hbm.at[pl.ds((start + j) * 64, 64)], wsems[k])

    rd(0, 0).start()
    rd(1, 1).start()

    def body(t, carry):
        for k in range(2):
            j = 2 * t + k
            rd(j, k).wait()

            @pl.when(t > 0)
            def _():
                wr(j - 2, k).wait()

            @plsc.parallel_loop(0, EMBED_DIM, 1, unroll=4)
            def _(d):
                for g in range(8):
                    val = srcs[k][d, pl.ds(16 * g, 16)]
                    plsc.store_scatter(dsts[k], [cA[g], cB[g] + d], val)
            wr(j, k).start()

            @pl.when(t < (_TPW // 2) - 1)
            def _():
                rd(j + 2, k).start()
        return carry

    lax.fori_loop(0, _TPW // 2, body, 0)
    wr(_TPW - 2, 0).wait()
    wr(_TPW - 1, 1).wait()

    @pl.when(wid == _NW - 1)
    def _():
        pltpu.sync_copy(tail_hbm, tailv)
        pltpu.sync_copy(tailv, scratch_hbm.at[pl.ds(_NTILE_A * 64, _TAIL_PAIRS)])


def _gather_kernel(scratch_hbm, idxt_hbm, out_hbm,
                   idx_v, i0, i1, i2, i3, g0, g1, g2, g3, t0, t1,
                   gs0, gs1, gs2, gs3, ts0, ts1):
    idx2 = (i0, i1, i2, i3)
    gbuf = (g0, g1, g2, g3)
    tbuf = (t0, t1)
    gsems = (gs0, gs1, gs2, gs3)
    tsems = (ts0, ts1)
    wid = lax.axis_index("s") * _NC + lax.axis_index("c")
    iota = lax.iota(jnp.int32, 16)
    idxL = tuple(16 * g + iota for g in range(8))

    pltpu.sync_copy(idxt_hbm.at[:, pl.ds(wid * 128, 128)], idx_v)

    def prep_and_fire(i, b):
        for c in range(8):
            idx2[b][pl.ds(16 * c, 16)] = lax.shift_right_logical(
                idx_v[i, pl.ds(16 * c, 16)], 1)
        pltpu.make_async_copy(scratch_hbm.at[idx2[b]], gbuf[b], gsems[b]).start()

    def out_tile(i, tb):
        for p in range(8):
            pltpu.make_async_copy(tbuf[tb].at[pl.ds(8 * p, 8), :],
                                  out_hbm.at[i, p, wid], tsems[tb]).start()

    def drain_out(i, tb):
        for p in range(8):
            pltpu.make_async_copy(tbuf[tb].at[pl.ds(8 * p, 8), :],
                                  out_hbm.at[i, p, wid], tsems[tb]).wait()

    for b in range(_NBUF):
        prep_and_fire(b, b)

    def body(t, carry):
        for b in range(_NBUF):
            i = _NBUF * t + b
            tb = b % 2
            pltpu.make_async_copy(scratch_hbm.at[idx2[b]], gbuf[b], gsems[b]).wait()

            if b >= 2:
                drain_out(i - 2, tb)
            else:
                @pl.when(t > 0)
                def _():
                    drain_out(i - 2, tb)

            halves = []
            for g in range(8):
                halves.append(
                    lax.shift_left(jnp.bitwise_and(idx_v[i, pl.ds(16 * g, 16)], 1), 6))

            @plsc.parallel_loop(0, EMBED_DIM, 1, unroll=4)
            def _(d):
                for g in range(8):
                    val = plsc.load_gather(gbuf[b], [idxL[g], halves[g] + d])
                    tbuf[tb][d, pl.ds(16 * g, 16)] = val
            out_tile(i, tb)

            @pl.when(t < (_NSEQ // _NBUF) - 1)
            def _():
                prep_and_fire(i + _NBUF, b)
        return carry

    lax.fori_loop(0, _NSEQ // _NBUF, body, 0)
    drain_out(_NSEQ - 2, 0)
    drain_out(_NSEQ - 1, 1)


@jax.jit
def kernel(input_sentence, table):
    mesh = plsc.VectorSubcoreMesh(core_axis_name="c", subcore_axis_name="s")
    cparams = pltpu.CompilerParams(use_tc_tiling_on_sc=True, needs_layout_passes=False)

    table_t = table.T
    tail = table[_NTILE_A * 128:].reshape(_TAIL_PAIRS, _LANES)
    idxt = input_sentence.T.astype(jnp.int32)

    scratch = pl.kernel(
        _relayout_kernel,
        out_type=jax.ShapeDtypeStruct((_PAIRS, _LANES), jnp.float32),
        mesh=mesh,
        scratch_types=(
            [pltpu.VMEM((EMBED_DIM, _LANES), jnp.float32) for _ in range(4)]
            + [pltpu.VMEM((_TAIL_PAIRS, _LANES), jnp.float32)]
            + [pltpu.SemaphoreType.DMA for _ in range(4)]
        ),
        compiler_params=cparams,
    )(table_t, tail)

    out5d = pl.kernel(
        _gather_kernel,
        out_type=jax.ShapeDtypeStruct((SEQ, 8, 32, 8, _LANES), jnp.float32),
        mesh=mesh,
        scratch_types=(
            [pltpu.VMEM((SEQ, _LANES), jnp.int32)]
            + [pltpu.VMEM((_LANES,), jnp.int32) for _ in range(_NBUF)]
            + [pltpu.VMEM((_LANES, _LANES), jnp.float32) for _ in range(_NBUF)]
            + [pltpu.VMEM((EMBED_DIM, _LANES), jnp.float32) for _ in range(2)]
            + [pltpu.SemaphoreType.DMA for _ in range(_NBUF + 2)]
        ),
        compiler_params=cparams,
    )(scratch, idxt)

    return out5d.transpose(2, 4, 0, 1, 3).reshape(BATCH, SEQ, EMBED_DIM)
